# Initial kernel scaffold; baseline (speedup 1.0000x reference)
import jax
import jax.numpy as jnp
from jax.experimental import pallas as pl


def kernel(token_ids, weight):
    raise NotImplementedError("write your pallas kernel here")


# speedup vs baseline: 1.6839x; 1.6839x over previous
import functools

import jax
import jax.numpy as jnp
from jax import lax
from jax.experimental import pallas as pl
from jax.experimental.pallas import tpu as pltpu
from jax.experimental.pallas import tpu_sc as plsc

NUM_EMB = 1_000_000
DIM = 64
BATCH = 16384 * 50

_info = plsc.get_sparse_core_info()
_NC, _NS = _info.num_cores, _info.num_subcores
NW = _NC * _NS
ROWS_PER_W = BATCH // NW
GCH = 128
G_PER_W = ROWS_PER_W // GCH

_mesh = plsc.VectorSubcoreMesh(core_axis_name="c", subcore_axis_name="s")


@functools.partial(
    pl.kernel,
    mesh=_mesh,
    out_type=jax.ShapeDtypeStruct((BATCH, DIM), jnp.float32),
    scratch_types=[
        pltpu.VMEM((G_PER_W, GCH), jnp.int32),
        pltpu.VMEM((GCH, DIM), jnp.float32),
        pltpu.SemaphoreType.DMA,
    ],
    compiler_params=pltpu.CompilerParams(use_tc_tiling_on_sc=False),
)
def _emb_lookup(idx_hbm, table_hbm, out_hbm, idx_v, rows_v, sem):
    wid = lax.axis_index("s") * _NC + lax.axis_index("c")
    gbase = wid * G_PER_W
    pltpu.sync_copy(idx_hbm.at[pl.ds(gbase, G_PER_W)], idx_v)

    def body(j, carry):
        pltpu.async_copy(table_hbm.at[idx_v.at[j]], rows_v, sem).wait()
        pltpu.sync_copy(rows_v, out_hbm.at[pl.ds((gbase + j) * GCH, GCH)])
        return carry

    lax.fori_loop(0, G_PER_W, body, 0)


def kernel(token_ids, weight):
    idx = token_ids.reshape(NW * G_PER_W, GCH).astype(jnp.int32)
    out = _emb_lookup(idx, weight)
    return out.reshape(token_ids.shape[0], token_ids.shape[1], DIM)


# speedup vs baseline: 1.8719x; 1.1116x over previous
import functools

import jax
import jax.numpy as jnp
from jax import lax
from jax.experimental import pallas as pl
from jax.experimental.pallas import tpu as pltpu
from jax.experimental.pallas import tpu_sc as plsc

NUM_EMB = 1_000_000
DIM = 64
BATCH = 16384 * 50

_info = plsc.get_sparse_core_info()
_NC, _NS = _info.num_cores, _info.num_subcores
NW = _NC * _NS
ROWS_PER_W = BATCH // NW
GCH = 128
K = 4
SCH = K * GCH
S = ROWS_PER_W // SCH
G_PER_W = ROWS_PER_W // GCH

_mesh = plsc.VectorSubcoreMesh(core_axis_name="c", subcore_axis_name="s")


@functools.partial(
    pl.kernel,
    mesh=_mesh,
    out_type=jax.ShapeDtypeStruct((BATCH, DIM), jnp.float32),
    scratch_types=[
        pltpu.VMEM((G_PER_W, GCH), jnp.int32),
        pltpu.VMEM((SCH, DIM), jnp.float32),
        pltpu.VMEM((SCH, DIM), jnp.float32),
        pltpu.SemaphoreType.DMA,
        pltpu.SemaphoreType.DMA,
        pltpu.SemaphoreType.DMA,
    ],
    compiler_params=pltpu.CompilerParams(use_tc_tiling_on_sc=False),
)
def _emb_lookup(idx_hbm, table_hbm, out_hbm, idx_v, buf_a, buf_b,
                sem_g, sem_a, sem_b):
    wid = lax.axis_index("s") * _NC + lax.axis_index("c")
    gbase = wid * G_PER_W
    rbase = wid * ROWS_PER_W
    pltpu.sync_copy(idx_hbm.at[pl.ds(gbase, G_PER_W)], idx_v)

    def fire_gathers(s, buf):
        for k in range(K):
            pltpu.async_copy(
                table_hbm.at[idx_v.at[s * K + k]],
                buf.at[pl.ds(k * GCH, GCH)],
                sem_g,
            )

    def wait_gathers(buf):
        pltpu.make_async_copy(table_hbm.at[pl.ds(0, SCH)], buf, sem_g).wait()

    def fire_out(s, buf, sem):
        pltpu.async_copy(buf, out_hbm.at[pl.ds(rbase + s * SCH, SCH)], sem)

    def wait_out(buf, sem):
        pltpu.make_async_copy(buf, out_hbm.at[pl.ds(rbase, SCH)], sem).wait()

    fire_gathers(0, buf_a)
    wait_gathers(buf_a)
    fire_out(0, buf_a, sem_a)
    fire_gathers(1, buf_b)

    def body(t, carry):
        s1 = 2 * t + 1
        wait_gathers(buf_b)
        fire_out(s1, buf_b, sem_b)
        wait_out(buf_a, sem_a)
        fire_gathers(s1 + 1, buf_a)

        s2 = 2 * t + 2
        wait_gathers(buf_a)
        fire_out(s2, buf_a, sem_a)
        wait_out(buf_b, sem_b)
        fire_gathers(s2 + 1, buf_b)
        return carry

    lax.fori_loop(0, (S - 2) // 2, body, 0)

    wait_gathers(buf_b)
    fire_out(S - 1, buf_b, sem_b)
    wait_out(buf_a, sem_a)
    wait_out(buf_b, sem_b)


def kernel(token_ids, weight):
    idx = token_ids.reshape(NW * G_PER_W, GCH).astype(jnp.int32)
    out = _emb_lookup(idx, weight)
    return out.reshape(token_ids.shape[0], token_ids.shape[1], DIM)


# speedup vs baseline: 1.8779x; 1.0032x over previous
import functools

import jax
import jax.numpy as jnp
from jax import lax
from jax.experimental import pallas as pl
from jax.experimental.pallas import tpu as pltpu
from jax.experimental.pallas import tpu_sc as plsc

NUM_EMB = 1_000_000
DIM = 64
BATCH = 16384 * 50

_info = plsc.get_sparse_core_info()
_NC, _NS = _info.num_cores, _info.num_subcores
NW = _NC * _NS
ROWS_PER_W = BATCH // NW
GCH = 128
K = 4
SCH = K * GCH
S = ROWS_PER_W // SCH
G_PER_W = ROWS_PER_W // GCH
NBUF = 3

_mesh = plsc.VectorSubcoreMesh(core_axis_name="c", subcore_axis_name="s")


@functools.partial(
    pl.kernel,
    mesh=_mesh,
    out_type=jax.ShapeDtypeStruct((BATCH, DIM), jnp.float32),
    scratch_types=[
        pltpu.VMEM((G_PER_W, GCH), jnp.int32),
        pltpu.VMEM((SCH, DIM), jnp.float32),
        pltpu.VMEM((SCH, DIM), jnp.float32),
        pltpu.VMEM((SCH, DIM), jnp.float32),
        pltpu.SemaphoreType.DMA,
        pltpu.SemaphoreType.DMA,
        pltpu.SemaphoreType.DMA,
        pltpu.SemaphoreType.DMA,
    ],
    compiler_params=pltpu.CompilerParams(use_tc_tiling_on_sc=False),
)
def _emb_lookup(idx_hbm, table_hbm, out_hbm, idx_v, buf0, buf1, buf2,
                sem_g, sem0, sem1, sem2):
    bufs = (buf0, buf1, buf2)
    sems = (sem0, sem1, sem2)
    wid = lax.axis_index("s") * _NC + lax.axis_index("c")
    gbase = wid * G_PER_W
    rbase = wid * ROWS_PER_W
    pltpu.sync_copy(idx_hbm.at[pl.ds(gbase, G_PER_W)], idx_v)

    def fire_gathers(s, b):
        for k in range(K):
            pltpu.async_copy(
                table_hbm.at[idx_v.at[s * K + k]],
                bufs[b].at[pl.ds(k * GCH, GCH)],
                sem_g,
            )

    def wait_gathers(b):
        pltpu.make_async_copy(
            table_hbm.at[pl.ds(0, SCH)], bufs[b], sem_g).wait()

    def fire_out(s, b):
        pltpu.async_copy(
            bufs[b], out_hbm.at[pl.ds(rbase + s * SCH, SCH)], sems[b])

    def wait_out(b):
        pltpu.make_async_copy(
            bufs[b], out_hbm.at[pl.ds(rbase, SCH)], sems[b]).wait()

    def step(s, b, fire_ahead=True):
        wait_gathers(b)
        fire_out(s, b)
        if fire_ahead:
            nb = (s + 2) % NBUF if isinstance(s, int) else (b + 2) % NBUF
            wait_out(nb)
            fire_gathers(s + 2, nb)

    fire_gathers(0, 0)
    fire_gathers(1, 1)
    wait_gathers(0)
    fire_out(0, 0)
    fire_gathers(2, 2)

    def body(t, carry):
        s = 3 * t + 1
        step(s, 1)
        step(s + 1, 2)
        step(s + 2, 0)
        return carry

    lax.fori_loop(0, (S - 5) // 3, body, 0)

    step(S - 4, (S - 4) % NBUF)
    step(S - 3, (S - 3) % NBUF)
    step(S - 2, (S - 2) % NBUF, fire_ahead=False)
    step(S - 1, (S - 1) % NBUF, fire_ahead=False)
    wait_out((S - 3) % NBUF)
    wait_out((S - 2) % NBUF)
    wait_out((S - 1) % NBUF)


def kernel(token_ids, weight):
    idx = token_ids.reshape(NW * G_PER_W, GCH).astype(jnp.int32)
    out = _emb_lookup(idx, weight)
    return out.reshape(token_ids.shape[0], token_ids.shape[1], DIM)
